# Initial kernel scaffold; baseline (speedup 1.0000x reference)
import jax
import jax.numpy as jnp
from jax.experimental import pallas as pl


def kernel(x, edge_index, W1, a_src1, a_dst1, b1, W2, a_src2, a_dst2, b2, W3, b3):
    raise NotImplementedError("write your pallas kernel here")


# speedup vs baseline: 1.2072x; 1.2072x over previous
import functools

import jax
import jax.numpy as jnp
from jax.experimental import pallas as pl

_N = 10000
_E = 320000
_F_IN = 128
_HID = 64
_H1 = 8
_OUT2 = 32
_NC = 16


def _dense1_body(x_ref, w_ref, aw_ref, h_ref, al_ref):
    h = jnp.dot(x_ref[...], w_ref[...], preferred_element_type=jnp.float32)
    h_ref[...] = h
    al_ref[...] = jnp.dot(h, aw_ref[...], preferred_element_type=jnp.float32)


def _dense1(x, W1, aw):
    n, f = x.shape
    k = W1.shape[1]
    bn = 1000
    grid = (n // bn,)
    return pl.pallas_call(
        _dense1_body,
        grid=grid,
        in_specs=[
            pl.BlockSpec((bn, f), lambda i: (i, 0)),
            pl.BlockSpec((f, k), lambda i: (0, 0)),
            pl.BlockSpec((k, aw.shape[1]), lambda i: (0, 0)),
        ],
        out_specs=[
            pl.BlockSpec((bn, k), lambda i: (i, 0)),
            pl.BlockSpec((bn, aw.shape[1]), lambda i: (i, 0)),
        ],
        out_shape=[
            jax.ShapeDtypeStruct((n, k), jnp.float32),
            jax.ShapeDtypeStruct((n, aw.shape[1]), jnp.float32),
        ],
    )(x, W1, aw)


def _logit_weights(a_s, a_d):
    heads, ch = a_s.shape
    eye = jnp.eye(heads, dtype=a_s.dtype)
    ws = (a_s[:, :, None] * eye[:, None, :]).reshape(heads * ch, heads)
    wd = (a_d[:, :, None] * eye[:, None, :]).reshape(heads * ch, heads)
    return jnp.concatenate([ws, wd], axis=1)


def _lrelu(v):
    return jnp.where(v > 0, v, 0.2 * v)


def _gat_sparse(h, al, src, dst, heads, ch):
    n = h.shape[0]
    a_s, a_d = al[:, :heads], al[:, heads:]
    ex = jnp.exp(_lrelu(a_s[src] + a_d[dst]))
    ex_self = jnp.exp(_lrelu(a_s + a_d))
    den = jax.ops.segment_sum(ex, dst, num_segments=n) + ex_self
    coef = ex / (den[dst] + 1e-16)
    hh = h.reshape(n, heads, ch)
    msg = hh[src] * coef[:, :, None]
    out = jax.ops.segment_sum(msg, dst, num_segments=n)
    out = out + (ex_self / (den + 1e-16))[:, :, None] * hh
    return out.reshape(n, heads * ch)


def kernel(x, edge_index, W1, a_src1, a_dst1, b1, W2, a_src2, a_dst2, b2, W3, b3):
    n = x.shape[0]
    src, dst = edge_index[0], edge_index[1]

    aw1 = _logit_weights(a_src1, a_dst1)
    h1, al1 = _dense1(x, W1, aw1)
    g1 = jax.nn.elu(_gat_sparse(h1, al1, src, dst, _H1, _HID) + b1[None, :])

    aw2 = _logit_weights(a_src2, a_dst2)
    h2, al2 = _dense1(g1, W2, aw2)
    g2 = jax.nn.elu(_gat_sparse(h2, al2, src, dst, 1, _OUT2) + b2[None, :])

    h3 = g2 @ W3
    ones = jnp.ones(src.shape[0], dtype=x.dtype)
    deg = jax.ops.segment_sum(ones, dst, num_segments=n) + 1.0
    dinv = deg ** -0.5
    g = dinv[:, None] * h3
    acc = jax.ops.segment_sum(g[src], dst, num_segments=n)
    out = dinv[:, None] * acc + dinv[:, None] ** 2 * h3 + b3[None, :]
    return out


# speedup vs baseline: 1.2293x; 1.0183x over previous
import functools

import jax
import jax.numpy as jnp
from jax import lax
from jax.experimental import pallas as pl
from jax.experimental.pallas import tpu as pltpu
from jax.experimental.pallas import tpu_sc as plsc

_N = 10000
_E = 320000
_F_IN = 128
_HID = 64
_H1 = 8
_OUT2 = 32
_NC = 16

_SC_CORES = 2
_SC_TILES = 16
_L = 16


def _zero_rows(ref, nrows):
    z = jnp.zeros((_L,), jnp.float32)

    def body(i, _):
        ref[i] = z
        return 0

    lax.fori_loop(0, nrows, body, 0)


def _fill_iota(ref, n, base):
    nfull = n // _L

    def body(i, _):
        ref[pl.ds(i * _L, _L)] = lax.iota(jnp.int32, _L) + (base + i * _L)
        return 0

    lax.fori_loop(0, nfull, body, 0)
    rem = n - nfull * _L
    if rem > 0:
        off = n - _L
        ref[pl.ds(off, _L)] = lax.iota(jnp.int32, _L) + (base + off)


def _gcn_gather_kernel(g_hbm, src_hbm, dst_hbm, out_hbm,
                       table, pend_src, pend_ld, srcv, dstv, rows, idxv,
                       spmem, sem):
    c = lax.axis_index("c")
    s = lax.axis_index("s")
    r = s // 4
    j = s % 4
    RNG = 2500
    ESH = _E // 8
    B = 800
    G = 128
    lo = r * RNG
    ebase = c * (_E // 2) + j * ESH

    _zero_rows(table, RNG)
    zi = jnp.zeros((_L,), jnp.int32)

    def zp_body(i, _):
        pend_src[pl.ds(i * _L, _L)] = zi
        pend_ld[pl.ds(i * _L, _L)] = zi
        return 0

    lax.fori_loop(0, (B + _L) // _L, zp_body, 0)
    zrows = 1000

    @pl.when(s < 10)
    def _():
        pltpu.sync_copy(table.at[pl.ds(0, zrows)],
                        spmem.at[pl.ds(s * zrows, zrows)])

    plsc.subcore_barrier()

    def chunk_body(k, _):
        pltpu.sync_copy(src_hbm.at[pl.ds(ebase + k * B, B)], srcv)
        pltpu.sync_copy(dst_hbm.at[pl.ds(ebase + k * B, B)], dstv)

        def scan_body(v, cnt):
            d = dstv[pl.ds(v * _L, _L)]
            m = (d >= lo) & (d < lo + RNG)
            csum = plsc.cumsum(m.astype(jnp.int32))
            pos = cnt + csum - 1
            plsc.store_scatter(pend_ld, [pos], d - lo, mask=m)
            plsc.store_scatter(pend_src, [pos], srcv[pl.ds(v * _L, _L)],
                               mask=m)
            return cnt + jnp.max(csum)

        cnt = lax.fori_loop(0, B // _L, scan_body, jnp.int32(0))

        def flush_body(b, _):
            off = b * G
            pltpu.async_copy(g_hbm.at[pend_src.at[pl.ds(off, G)]], rows,
                             sem).wait()
            nin = jnp.minimum(cnt - off, G)

            def acc_body(i, _):
                ld = pend_ld[pl.ds(off + i, _L)][0]
                table[ld] = table[ld] + rows[i]
                return 0

            lax.fori_loop(0, nin, acc_body, 0)
            return 0

        lax.fori_loop(0, (cnt + G - 1) // G, flush_body, 0)
        return 0

    lax.fori_loop(0, ESH // B, chunk_body, 0)

    _fill_iota(idxv, RNG, lo)
    pltpu.sync_copy(table, spmem.at[idxv], add=True)
    plsc.subcore_barrier()

    @pl.when(s < 10)
    def _():
        pltpu.sync_copy(spmem.at[pl.ds(s * zrows, zrows)],
                        out_hbm.at[c, pl.ds(s * zrows, zrows)])


def _gcn_gather(g, src, dst):
    mesh = plsc.VectorSubcoreMesh(core_axis_name="c", subcore_axis_name="s")
    B = 800
    G = 128
    RNG = 2500
    f = pl.kernel(
        _gcn_gather_kernel,
        mesh=mesh,
        compiler_params=pltpu.CompilerParams(
            use_tc_tiling_on_sc=False, needs_layout_passes=False),
        out_type=jax.ShapeDtypeStruct((_SC_CORES, _N, 16), jnp.float32),
        scratch_types=[
            pltpu.VMEM((RNG, 16), jnp.float32),
            pltpu.VMEM((B + _L,), jnp.int32),
            pltpu.VMEM((B + _L,), jnp.int32),
            pltpu.VMEM((B,), jnp.int32),
            pltpu.VMEM((B,), jnp.int32),
            pltpu.VMEM((G, 16), jnp.float32),
            pltpu.VMEM((RNG,), jnp.int32),
            pltpu.VMEM_SHARED((_N, 16), jnp.float32),
            pltpu.SemaphoreType.DMA,
        ],
    )
    return f(g, src, dst)


def _dense1_body(x_ref, w_ref, aw_ref, h_ref, al_ref):
    h = jnp.dot(x_ref[...], w_ref[...], preferred_element_type=jnp.float32)
    h_ref[...] = h
    al_ref[...] = jnp.dot(h, aw_ref[...], preferred_element_type=jnp.float32)


def _dense1(x, W1, aw):
    n, f = x.shape
    k = W1.shape[1]
    bn = 1000
    grid = (n // bn,)
    return pl.pallas_call(
        _dense1_body,
        grid=grid,
        in_specs=[
            pl.BlockSpec((bn, f), lambda i: (i, 0)),
            pl.BlockSpec((f, k), lambda i: (0, 0)),
            pl.BlockSpec((k, aw.shape[1]), lambda i: (0, 0)),
        ],
        out_specs=[
            pl.BlockSpec((bn, k), lambda i: (i, 0)),
            pl.BlockSpec((bn, aw.shape[1]), lambda i: (i, 0)),
        ],
        out_shape=[
            jax.ShapeDtypeStruct((n, k), jnp.float32),
            jax.ShapeDtypeStruct((n, aw.shape[1]), jnp.float32),
        ],
    )(x, W1, aw)


def _logit_weights(a_s, a_d):
    heads, ch = a_s.shape
    eye = jnp.eye(heads, dtype=a_s.dtype)
    ws = (a_s[:, :, None] * eye[:, None, :]).reshape(heads * ch, heads)
    wd = (a_d[:, :, None] * eye[:, None, :]).reshape(heads * ch, heads)
    return jnp.concatenate([ws, wd], axis=1)


def _lrelu(v):
    return jnp.where(v > 0, v, 0.2 * v)


def _gat_sparse(h, al, src, dst, heads, ch):
    n = h.shape[0]
    a_s, a_d = al[:, :heads], al[:, heads:]
    ex = jnp.exp(_lrelu(a_s[src] + a_d[dst]))
    ex_self = jnp.exp(_lrelu(a_s + a_d))
    den = jax.ops.segment_sum(ex, dst, num_segments=n) + ex_self
    coef = ex / (den[dst] + 1e-16)
    hh = h.reshape(n, heads, ch)
    msg = hh[src] * coef[:, :, None]
    out = jax.ops.segment_sum(msg, dst, num_segments=n)
    out = out + (ex_self / (den + 1e-16))[:, :, None] * hh
    return out.reshape(n, heads * ch)


def kernel(x, edge_index, W1, a_src1, a_dst1, b1, W2, a_src2, a_dst2, b2, W3, b3):
    n = x.shape[0]
    src, dst = edge_index[0], edge_index[1]

    aw1 = _logit_weights(a_src1, a_dst1)
    h1, al1 = _dense1(x, W1, aw1)
    g1 = jax.nn.elu(_gat_sparse(h1, al1, src, dst, _H1, _HID) + b1[None, :])

    aw2 = _logit_weights(a_src2, a_dst2)
    h2, al2 = _dense1(g1, W2, aw2)
    g2 = jax.nn.elu(_gat_sparse(h2, al2, src, dst, 1, _OUT2) + b2[None, :])

    h3 = g2 @ W3
    ones = jnp.ones(src.shape[0], dtype=x.dtype)
    deg = jax.ops.segment_sum(ones, dst, num_segments=n) + 1.0
    dinv = deg ** -0.5
    g = dinv[:, None] * h3
    accp = _gcn_gather(g, src, dst)
    acc = accp[0] + accp[1]
    out = dinv[:, None] * acc + dinv[:, None] ** 2 * h3 + b3[None, :]
    return out
